# baseline (speedup 1.0000x reference)
import functools

import jax
import jax.numpy as jnp
from jax import lax
from jax.experimental import pallas as pl
from jax.experimental.pallas import tpu as pltpu
from jax.experimental.pallas import tpu_sc as plsc

N = 100000
D = 128
DEG = 16
B = 1024
NEG = 10
Q = 2 * B + NEG
NW = 32
QP = 2304
BPW = QP // NW
CH = 12
NCH = BPW // CH
NPACK = N // 2


def _sc_gather_kernel(nodes_hbm, neighbors_hbm, feat_hbm, degp_hbm,
                      s_out, degr_out, nf_out,
                      nodes_v, nb_v, nb1d, degp_v, degv, rows_a,
                      rows_b, nf_v, s_v, sem_nf, sem_deg, sem_a, sem_b):
    wid = lax.axis_index("s") * 2 + lax.axis_index("c")
    base = wid * BPW

    deg_stage = pltpu.async_copy(degp_hbm, degp_v, sem_deg)
    nodes_v[pl.ds(64, 16)] = jnp.zeros((16,), jnp.int32)
    pltpu.sync_copy(nodes_hbm.at[pl.ds(base, BPW)],
                    nodes_v.at[pl.ds(0, BPW)])

    starts = []
    for g in range((BPW + 15) // 16):
        nv = nodes_v[pl.ds(g * 16, 16)]
        for j in range(16):
            q = g * 16 + j
            if q >= BPW:
                break
            starts.append(pltpu.async_copy(
                neighbors_hbm.at[pl.ds(nv[j], 1)],
                nb_v.at[pl.ds(q, 1)], sem_a))
    for c in starts:
        c.wait()

    def flatten_row(q, _):
        nb1d[pl.ds(q * DEG, DEG)] = nb_v[q, :]
        return 0

    lax.fori_loop(0, BPW, flatten_row, 0)

    rows = (rows_a, rows_b)
    csems = (sem_a, sem_b)

    def start(ci):
        return pltpu.async_copy(
            feat_hbm.at[nb1d.at[pl.ds(ci * CH * DEG, CH * DEG)]],
            rows[ci % 2], csems[ci % 2])

    cur = start(0)
    nf_copy = pltpu.async_copy(feat_hbm.at[nodes_v.at[pl.ds(0, BPW)]],
                               nf_v, sem_nf)
    for ci in range(NCH):
        nxt = start(ci + 1) if ci + 1 < NCH else None
        cur.wait()
        rv = rows[ci % 2]

        def reduce_one(q, _):
            for c in range(D // 16):
                acc = rv[q * DEG, pl.ds(c * 16, 16)]
                for k in range(1, DEG):
                    acc = acc + rv[q * DEG + k, pl.ds(c * 16, 16)]
                s_v[ci * CH + q, pl.ds(c * 16, 16)] = acc
            return 0

        lax.fori_loop(0, CH, reduce_one, 0)
        cur = nxt

    pltpu.sync_copy(s_v, s_out.at[pl.ds(base, BPW)])

    nf_copy.wait()
    pltpu.sync_copy(nf_v, nf_out.at[pl.ds(base, BPW)])

    deg_stage.wait()

    def pick_deg(q, _):
        v = nb_v[q, :]
        word = plsc.load_gather(degp_v, [lax.shift_right_logical(v, 1)])
        shift = jnp.where(lax.bitwise_and(v, 1) == 0, 16, 0)
        bits = lax.bitwise_and(
            lax.shift_left(word, shift),
            jnp.full((16,), -65536, dtype=jnp.int32))
        degv[pl.ds(q * DEG, DEG)] = plsc.bitcast(bits, jnp.float32)
        return 0

    lax.fori_loop(0, BPW, pick_deg, 0)
    pltpu.sync_copy(degv, degr_out.at[pl.ds(base * DEG, BPW * DEG)])


@functools.lru_cache(maxsize=None)
def _build_sc_gather():
    return pl.kernel(
        _sc_gather_kernel,
        out_type=[
            jax.ShapeDtypeStruct((QP, D), jnp.float32),
            jax.ShapeDtypeStruct((QP * DEG,), jnp.float32),
            jax.ShapeDtypeStruct((QP, D), jnp.float32),
        ],
        mesh=plsc.VectorSubcoreMesh(core_axis_name="c",
                                    subcore_axis_name="s"),
        compiler_params=pltpu.CompilerParams(needs_layout_passes=False),
        scratch_types=[
            pltpu.VMEM((80,), jnp.int32),
            pltpu.VMEM((BPW, DEG), jnp.int32),
            pltpu.VMEM((BPW * DEG,), jnp.int32),
            pltpu.VMEM((NPACK,), jnp.int32),
            pltpu.VMEM((BPW * DEG,), jnp.float32),
            pltpu.VMEM((CH * DEG, D), jnp.float32),
            pltpu.VMEM((CH * DEG, D), jnp.float32),
            pltpu.VMEM((BPW, D), jnp.float32),
            pltpu.VMEM((BPW, D), jnp.float32),
            pltpu.SemaphoreType.DMA,
            pltpu.SemaphoreType.DMA,
            pltpu.SemaphoreType.DMA,
            pltpu.SemaphoreType.DMA,
        ],
    )


def _tc_tail_kernel(s_ref, deg_ref, nf_ref, w1_ref, b1_ref, w2_ref, b2_ref,
                    o_ref):
    g = jnp.sum(deg_ref[...], axis=1, keepdims=True)
    w1s = jnp.sum(w1_ref[:, D:], axis=1)
    ne = lax.dot_general(s_ref[...], w1_ref[:, :D],
                         (((1,), (1,)), ((), ())),
                         preferred_element_type=jnp.float32)
    ne = ne + g * w1s[None, :] + float(DEG) * b1_ref[...]
    f = lax.dot_general(nf_ref[...], w2_ref[:, :D],
                        (((1,), (1,)), ((), ())),
                        preferred_element_type=jnp.float32)
    f = f + lax.dot_general(ne, w2_ref[:, D:],
                            (((1,), (1,)), ((), ())),
                            preferred_element_type=jnp.float32)
    f = f + b2_ref[...]
    n = jnp.sqrt(jnp.sum(f * f, axis=1, keepdims=True))
    o_ref[...] = f / jnp.maximum(n, 1e-12)


def kernel(inputs1, inputs2, neg, neighbors, feat_data, degrees, W1, b1, W2,
           b2):
    nodes = jnp.concatenate([inputs1, inputs2, neg]).astype(jnp.int32)
    nodes = jnp.pad(nodes, (0, QP - Q))
    degp = jax.lax.bitcast_convert_type(
        degrees.astype(jnp.bfloat16).reshape(NPACK, 2),
        jnp.int32).reshape(NPACK)

    s, degr, nf = _build_sc_gather()(nodes, neighbors.astype(jnp.int32),
                                     feat_data, degp)

    out = pl.pallas_call(
        _tc_tail_kernel,
        out_shape=jax.ShapeDtypeStruct((QP, D), jnp.float32),
    )(s, degr.reshape(QP, DEG), nf, W1, b1.reshape(1, D), W2,
      b2.reshape(1, D))

    return out[:B], out[B:2 * B], out[2 * B:Q]


# speedup vs baseline: 1.8783x; 1.8783x over previous
import functools

import jax
import jax.numpy as jnp
from jax import lax
from jax.experimental import pallas as pl
from jax.experimental.pallas import tpu as pltpu
from jax.experimental.pallas import tpu_sc as plsc

N = 100000
D = 128
DEG = 16
B = 1024
NEG = 10
Q = 2 * B + NEG
NW = 32
QP = 2304
BPW = QP // NW
CH = 9
NCH = BPW // CH
NPACK = N // 2


def _sc_gather_kernel(nodes_hbm, nbflat_hbm, feat_hbm, degp_hbm,
                      s_out, degr_out, nf_out,
                      nodes_v, idxnb, nb1d, degv, rows_a,
                      rows_b, nf_v, s_v, sem_nf, sem_deg, sem_a, sem_b):
    wid = lax.axis_index("s") * 2 + lax.axis_index("c")
    base = wid * BPW

    pltpu.sync_copy(nodes_hbm.at[pl.ds(base, BPW)],
                    nodes_v.at[pl.ds(0, BPW)])

    lanesN = lax.iota(jnp.int32, 16) * N
    for g in range((BPW + 15) // 16):
        nv = nodes_v[pl.ds(g * 16, 16)]
        for j in range(16):
            q = g * 16 + j
            if q >= BPW:
                break
            idxnb[pl.ds(q * DEG, DEG)] = nv[j] + lanesN
    pltpu.async_copy(nbflat_hbm.at[idxnb], nb1d, sem_a).wait()

    rows = (rows_a, rows_b)
    csems = (sem_a, sem_b)

    def start(ci):
        return pltpu.async_copy(
            feat_hbm.at[nb1d.at[pl.ds(ci * CH * DEG, CH * DEG)]],
            rows[ci % 2], csems[ci % 2])

    cur = start(0)
    nf_copy = pltpu.async_copy(feat_hbm.at[nodes_v.at[pl.ds(0, BPW)]],
                               nf_v, sem_nf)
    for ci in range(NCH):
        nxt = start(ci + 1) if ci + 1 < NCH else None
        cur.wait()
        rv = rows[ci % 2]

        def reduce_one(q, _):
            for c in range(D // 16):
                acc = rv[q * DEG, pl.ds(c * 16, 16)]
                for k in range(1, DEG):
                    acc = acc + rv[q * DEG + k, pl.ds(c * 16, 16)]
                s_v[ci * CH + q, pl.ds(c * 16, 16)] = acc
            return 0

        lax.fori_loop(0, CH, reduce_one, 0)
        cur = nxt

    pltpu.sync_copy(s_v, s_out.at[pl.ds(base, BPW)])

    nf_copy.wait()
    pltpu.sync_copy(nf_v, nf_out.at[pl.ds(base, BPW)])

    pltpu.async_copy(degp_hbm.at[nb1d], degv, sem_deg).wait()
    pltpu.sync_copy(degv, degr_out.at[pl.ds(base * DEG, BPW * DEG)])


@functools.lru_cache(maxsize=None)
def _build_sc_gather():
    return pl.kernel(
        _sc_gather_kernel,
        out_type=[
            jax.ShapeDtypeStruct((QP, D), jnp.float32),
            jax.ShapeDtypeStruct((QP * DEG,), jnp.float32),
            jax.ShapeDtypeStruct((QP, D), jnp.float32),
        ],
        mesh=plsc.VectorSubcoreMesh(core_axis_name="c",
                                    subcore_axis_name="s"),
        compiler_params=pltpu.CompilerParams(needs_layout_passes=False),
        scratch_types=[
            pltpu.VMEM((80,), jnp.int32),
            pltpu.VMEM((BPW * DEG,), jnp.int32),
            pltpu.VMEM((BPW * DEG,), jnp.int32),
            pltpu.VMEM((BPW * DEG,), jnp.float32),
            pltpu.VMEM((CH * DEG, D), jnp.float32),
            pltpu.VMEM((CH * DEG, D), jnp.float32),
            pltpu.VMEM((BPW, D), jnp.float32),
            pltpu.VMEM((BPW, D), jnp.float32),
            pltpu.SemaphoreType.DMA,
            pltpu.SemaphoreType.DMA,
            pltpu.SemaphoreType.DMA,
            pltpu.SemaphoreType.DMA,
        ],
    )


def _tc_tail_kernel(s_ref, deg_ref, nf_ref, w1_ref, b1_ref, w2_ref, b2_ref,
                    o_ref):
    g = jnp.sum(deg_ref[...], axis=1, keepdims=True)
    w1s = jnp.sum(w1_ref[:, D:], axis=1)
    ne = lax.dot_general(s_ref[...], w1_ref[:, :D],
                         (((1,), (1,)), ((), ())),
                         preferred_element_type=jnp.float32)
    ne = ne + g * w1s[None, :] + float(DEG) * b1_ref[...]
    f = lax.dot_general(nf_ref[...], w2_ref[:, :D],
                        (((1,), (1,)), ((), ())),
                        preferred_element_type=jnp.float32)
    f = f + lax.dot_general(ne, w2_ref[:, D:],
                            (((1,), (1,)), ((), ())),
                            preferred_element_type=jnp.float32)
    f = f + b2_ref[...]
    n = jnp.sqrt(jnp.sum(f * f, axis=1, keepdims=True))
    o_ref[...] = f / jnp.maximum(n, 1e-12)


def kernel(inputs1, inputs2, neg, neighbors, feat_data, degrees, W1, b1, W2,
           b2):
    nodes = jnp.concatenate([inputs1, inputs2, neg]).astype(jnp.int32)
    nodes = jnp.pad(nodes, (0, QP - Q))
    nbflat = neighbors.astype(jnp.int32).T.reshape(N * DEG)
    s, degr, nf = _build_sc_gather()(nodes, nbflat, feat_data, degrees)

    out = pl.pallas_call(
        _tc_tail_kernel,
        out_shape=jax.ShapeDtypeStruct((QP, D), jnp.float32),
    )(s, degr.reshape(QP, DEG), nf, W1, b1.reshape(1, D), W2,
      b2.reshape(1, D))

    return out[:B], out[B:2 * B], out[2 * B:Q]


# speedup vs baseline: 1.9373x; 1.0314x over previous
import functools

import jax
import jax.numpy as jnp
from jax import lax
from jax.experimental import pallas as pl
from jax.experimental.pallas import tpu as pltpu
from jax.experimental.pallas import tpu_sc as plsc

N = 100000
D = 128
DEG = 16
B = 1024
NEG = 10
Q = 2 * B + NEG
NW = 32
QP = 2304
BPW = QP // NW
CH = 9
NCH = BPW // CH


def _sc_gather_kernel(nodes_hbm, nbflat_hbm, feat_hbm, degp_hbm,
                      s_out, degr_out, nf_out,
                      nodes_v, idxnb, nb1d, degv, rows_a,
                      rows_b, nf_v, s_v, sem_nf, sem_deg, sem_a, sem_b):
    wid = lax.axis_index("s") * 2 + lax.axis_index("c")
    base = wid * BPW

    pltpu.sync_copy(nodes_hbm.at[pl.ds(base, BPW)],
                    nodes_v.at[pl.ds(0, BPW)])

    lanesN = lax.iota(jnp.int32, 16) * N
    for g in range((BPW + 15) // 16):
        nv = nodes_v[pl.ds(g * 16, 16)]
        for j in range(16):
            q = g * 16 + j
            if q >= BPW:
                break
            idxnb[pl.ds(q * DEG, DEG)] = nv[j] + lanesN
    pltpu.async_copy(nbflat_hbm.at[idxnb], nb1d, sem_a).wait()

    rows = (rows_a, rows_b)
    csems = (sem_a, sem_b)

    def start(ci):
        return pltpu.async_copy(
            feat_hbm.at[nb1d.at[pl.ds(ci * CH * DEG, CH * DEG)]],
            rows[ci % 2], csems[ci % 2])

    cur = start(0)
    nf_copy = pltpu.async_copy(feat_hbm.at[nodes_v.at[pl.ds(0, BPW)]],
                               nf_v, sem_nf)
    for ci in range(NCH):
        nxt = start(ci + 1) if ci + 1 < NCH else None
        cur.wait()
        rv = rows[ci % 2]

        def reduce_one(q, _):
            for c in range(D // 16):
                acc = rv[q * DEG, pl.ds(c * 16, 16)]
                for k in range(1, DEG):
                    acc = acc + rv[q * DEG + k, pl.ds(c * 16, 16)]
                s_v[ci * CH + q, pl.ds(c * 16, 16)] = acc
            return 0

        lax.fori_loop(0, CH, reduce_one, 0)
        cur = nxt

    pltpu.sync_copy(s_v, s_out.at[pl.ds(base, BPW)])

    nf_copy.wait()
    pltpu.sync_copy(nf_v, nf_out.at[pl.ds(base, BPW)])

    pltpu.async_copy(degp_hbm.at[nb1d], degv, sem_deg).wait()
    pltpu.sync_copy(degv, degr_out.at[pl.ds(base * DEG, BPW * DEG)])


@functools.lru_cache(maxsize=None)
def _build_sc_gather():
    return pl.kernel(
        _sc_gather_kernel,
        out_type=[
            jax.ShapeDtypeStruct((QP, D), jnp.float32),
            jax.ShapeDtypeStruct((QP * DEG,), jnp.float32),
            jax.ShapeDtypeStruct((QP, D), jnp.float32),
        ],
        mesh=plsc.VectorSubcoreMesh(core_axis_name="c",
                                    subcore_axis_name="s"),
        compiler_params=pltpu.CompilerParams(needs_layout_passes=False),
        scratch_types=[
            pltpu.VMEM((80,), jnp.int32),
            pltpu.VMEM((BPW * DEG,), jnp.int32),
            pltpu.VMEM((BPW * DEG,), jnp.int32),
            pltpu.VMEM((BPW * DEG,), jnp.float32),
            pltpu.VMEM((CH * DEG, D), jnp.float32),
            pltpu.VMEM((CH * DEG, D), jnp.float32),
            pltpu.VMEM((BPW, D), jnp.float32),
            pltpu.VMEM((BPW, D), jnp.float32),
            pltpu.SemaphoreType.DMA,
            pltpu.SemaphoreType.DMA,
            pltpu.SemaphoreType.DMA,
            pltpu.SemaphoreType.DMA,
        ],
    )


def _tc_tail_kernel(s_ref, deg_ref, nf_ref, w1_ref, b1_ref, w2_ref, b2_ref,
                    o1_ref, o2_ref, o3_ref):
    g = jnp.sum(deg_ref[...], axis=1, keepdims=True)
    w1s = jnp.sum(w1_ref[:, D:], axis=1)
    ne = lax.dot_general(s_ref[...], w1_ref[:, :D],
                         (((1,), (1,)), ((), ())),
                         preferred_element_type=jnp.float32)
    ne = ne + g * w1s[None, :] + float(DEG) * b1_ref[...]
    f = lax.dot_general(nf_ref[...], w2_ref[:, :D],
                        (((1,), (1,)), ((), ())),
                        preferred_element_type=jnp.float32)
    f = f + lax.dot_general(ne, w2_ref[:, D:],
                            (((1,), (1,)), ((), ())),
                            preferred_element_type=jnp.float32)
    f = f + b2_ref[...]
    n = jnp.sqrt(jnp.sum(f * f, axis=1, keepdims=True))
    f = f / jnp.maximum(n, 1e-12)
    o1_ref[...] = f[0:B]
    o2_ref[...] = f[B:2 * B]
    o3_ref[...] = f[2 * B:2 * B + 256]


def kernel(inputs1, inputs2, neg, neighbors, feat_data, degrees, W1, b1, W2,
           b2):
    nodes = jnp.concatenate([inputs1, inputs2, neg]).astype(jnp.int32)
    nodes = jnp.pad(nodes, (0, QP - Q))
    nbflat = neighbors.astype(jnp.int32).T.reshape(N * DEG)
    s, degr, nf = _build_sc_gather()(nodes, nbflat, feat_data, degrees)

    out1, out2, out3 = pl.pallas_call(
        _tc_tail_kernel,
        out_shape=[
            jax.ShapeDtypeStruct((B, D), jnp.float32),
            jax.ShapeDtypeStruct((B, D), jnp.float32),
            jax.ShapeDtypeStruct((256, D), jnp.float32),
        ],
    )(s, degr.reshape(QP, DEG), nf, W1, b1.reshape(1, D), W2,
      b2.reshape(1, D))

    return out1, out2, out3[:NEG]


# speedup vs baseline: 2.0371x; 1.0515x over previous
import functools

import jax
import jax.numpy as jnp
from jax import lax
from jax.experimental import pallas as pl
from jax.experimental.pallas import tpu as pltpu
from jax.experimental.pallas import tpu_sc as plsc

N = 100000
D = 128
DEG = 16
B = 1024
NEG = 10
Q = 2 * B + NEG
NW = 32
QP = 2304
BPW = QP // NW
CH = 18
NCH = BPW // CH


def _sc_gather_kernel(nodes_hbm, nbflat_hbm, feat_hbm, degp_hbm,
                      s_out, degr_out, nf_out,
                      nodes_v, idxnb, nb1d, degv, rows_a,
                      rows_b, nf_v, s_v, sem_nf, sem_deg, sem_a, sem_b):
    wid = lax.axis_index("s") * 2 + lax.axis_index("c")
    base = wid * BPW

    pltpu.sync_copy(nodes_hbm.at[pl.ds(base, BPW)],
                    nodes_v.at[pl.ds(0, BPW)])

    lanesN = lax.iota(jnp.int32, 16) * N
    for g in range((BPW + 15) // 16):
        nv = nodes_v[pl.ds(g * 16, 16)]
        for j in range(16):
            q = g * 16 + j
            if q >= BPW:
                break
            idxnb[pl.ds(q * DEG, DEG)] = nv[j] + lanesN
    pltpu.async_copy(nbflat_hbm.at[idxnb], nb1d, sem_a).wait()

    rows = (rows_a, rows_b)
    csems = (sem_a, sem_b)

    def start(ci):
        return pltpu.async_copy(
            feat_hbm.at[nb1d.at[pl.ds(ci * CH * DEG, CH * DEG)]],
            rows[ci % 2], csems[ci % 2])

    cur = start(0)
    nf_copy = pltpu.async_copy(feat_hbm.at[nodes_v.at[pl.ds(0, BPW)]],
                               nf_v, sem_nf)
    for ci in range(NCH):
        nxt = start(ci + 1) if ci + 1 < NCH else None
        cur.wait()
        rv = rows[ci % 2]

        def reduce_one(q, _):
            for c in range(D // 16):
                acc = rv[q * DEG, pl.ds(c * 16, 16)]
                for k in range(1, DEG):
                    acc = acc + rv[q * DEG + k, pl.ds(c * 16, 16)]
                s_v[ci * CH + q, pl.ds(c * 16, 16)] = acc
            return 0

        lax.fori_loop(0, CH, reduce_one, 0)
        cur = nxt

    pltpu.sync_copy(s_v, s_out.at[pl.ds(base, BPW)])

    nf_copy.wait()
    pltpu.sync_copy(nf_v, nf_out.at[pl.ds(base, BPW)])

    pltpu.async_copy(degp_hbm.at[nb1d], degv, sem_deg).wait()
    pltpu.sync_copy(degv, degr_out.at[pl.ds(base * DEG, BPW * DEG)])


@functools.lru_cache(maxsize=None)
def _build_sc_gather():
    return pl.kernel(
        _sc_gather_kernel,
        out_type=[
            jax.ShapeDtypeStruct((QP, D), jnp.float32),
            jax.ShapeDtypeStruct((QP * DEG,), jnp.float32),
            jax.ShapeDtypeStruct((QP, D), jnp.float32),
        ],
        mesh=plsc.VectorSubcoreMesh(core_axis_name="c",
                                    subcore_axis_name="s"),
        compiler_params=pltpu.CompilerParams(needs_layout_passes=False),
        scratch_types=[
            pltpu.VMEM((80,), jnp.int32),
            pltpu.VMEM((BPW * DEG,), jnp.int32),
            pltpu.VMEM((BPW * DEG,), jnp.int32),
            pltpu.VMEM((BPW * DEG,), jnp.float32),
            pltpu.VMEM((CH * DEG, D), jnp.float32),
            pltpu.VMEM((CH * DEG, D), jnp.float32),
            pltpu.VMEM((BPW, D), jnp.float32),
            pltpu.VMEM((BPW, D), jnp.float32),
            pltpu.SemaphoreType.DMA,
            pltpu.SemaphoreType.DMA,
            pltpu.SemaphoreType.DMA,
            pltpu.SemaphoreType.DMA,
        ],
    )


def _tc_tail_kernel(s_ref, deg_ref, nf_ref, w1_ref, b1_ref, w2_ref, b2_ref,
                    o1_ref, o2_ref, o3_ref):
    g = jnp.sum(deg_ref[...], axis=1, keepdims=True)
    w1s = jnp.sum(w1_ref[:, D:], axis=1)
    ne = lax.dot_general(s_ref[...], w1_ref[:, :D],
                         (((1,), (1,)), ((), ())),
                         preferred_element_type=jnp.float32)
    ne = ne + g * w1s[None, :] + float(DEG) * b1_ref[...]
    f = lax.dot_general(nf_ref[...], w2_ref[:, :D],
                        (((1,), (1,)), ((), ())),
                        preferred_element_type=jnp.float32)
    f = f + lax.dot_general(ne, w2_ref[:, D:],
                            (((1,), (1,)), ((), ())),
                            preferred_element_type=jnp.float32)
    f = f + b2_ref[...]
    n = jnp.sqrt(jnp.sum(f * f, axis=1, keepdims=True))
    f = f / jnp.maximum(n, 1e-12)
    o1_ref[...] = f[0:B]
    o2_ref[...] = f[B:2 * B]
    o3_ref[...] = f[2 * B:2 * B + 256]


def kernel(inputs1, inputs2, neg, neighbors, feat_data, degrees, W1, b1, W2,
           b2):
    nodes = jnp.concatenate([inputs1, inputs2, neg]).astype(jnp.int32)
    nodes = jnp.pad(nodes, (0, QP - Q))
    nbflat = neighbors.astype(jnp.int32).T.reshape(N * DEG)
    s, degr, nf = _build_sc_gather()(nodes, nbflat, feat_data, degrees)

    out1, out2, out3 = pl.pallas_call(
        _tc_tail_kernel,
        out_shape=[
            jax.ShapeDtypeStruct((B, D), jnp.float32),
            jax.ShapeDtypeStruct((B, D), jnp.float32),
            jax.ShapeDtypeStruct((256, D), jnp.float32),
        ],
    )(s, degr.reshape(QP, DEG), nf, W1, b1.reshape(1, D), W2,
      b2.reshape(1, D))

    return out1, out2, out3[:NEG]
